# baseline (speedup 1.0000x reference)
import functools

import jax
import jax.numpy as jnp
from jax import lax
from jax.experimental import pallas as pl
from jax.experimental.pallas import tpu as pltpu
from jax.experimental.pallas import tpu_sc as plsc

_NUM_WORKERS = 32
_LANES = 16
_NCHUNK = 7
_NBUF = 3


def _round16(x):
    return (x + _LANES - 1) // _LANES * _LANES


@functools.lru_cache(maxsize=None)
def _build_sc_exp_map(n: int):
    c_std = _round16(-(-n // _NUM_WORKERS))
    cw = -(-c_std // _NCHUNK)
    cw = _round16(cw)
    c_std = cw * _NCHUNK
    last_start = (_NUM_WORKERS - 1) * c_std
    c_last = n - last_start
    assert 0 < c_last <= c_std
    full_last = (c_last - 1) // cw
    cw_tail = c_last - full_last * cw
    assert 0 < cw_tail <= cw

    mesh = plsc.VectorSubcoreMesh(core_axis_name="c", subcore_axis_name="s")

    @functools.partial(
        pl.kernel,
        out_type=jax.ShapeDtypeStruct((n,), jnp.float32),
        mesh=mesh,
        scratch_types=[
            pltpu.VMEM((cw,), jnp.float32),
            pltpu.VMEM((cw,), jnp.float32),
            pltpu.VMEM((cw,), jnp.float32),
            pltpu.VMEM((_LANES,), jnp.float32),
            pltpu.SemaphoreType.DMA,
            pltpu.SemaphoreType.DMA,
            pltpu.SemaphoreType.DMA,
            pltpu.SemaphoreType.DMA,
            pltpu.SemaphoreType.DMA,
            pltpu.SemaphoreType.DMA,
        ],
    )
    def run(vals, scale, out, buf0, buf1, buf2, scale_v, si0, si1, si2, so0, so1, so2):
        bufs = (buf0, buf1, buf2)
        sem_in = (si0, si1, si2)
        sem_out = (so0, so1, so2)
        wid = lax.axis_index("c") * 16 + lax.axis_index("s")
        pltpu.sync_copy(scale, scale_v)
        s = scale_v[...]

        def pipeline(base, chunk_lens):
            nk = len(chunk_lens)

            def in_copy(k):
                b = k % _NBUF
                w = chunk_lens[k]
                return pltpu.make_async_copy(
                    vals.at[pl.ds(base + k * cw, w)],
                    bufs[b].at[pl.ds(0, w)],
                    sem_in[b],
                )

            def out_copy(k):
                b = k % _NBUF
                w = chunk_lens[k]
                return pltpu.make_async_copy(
                    bufs[b].at[pl.ds(0, w)],
                    out.at[pl.ds(base + k * cw, w)],
                    sem_out[b],
                )

            def compute(k):
                buf = bufs[k % _NBUF]
                w = _round16(chunk_lens[k])

                @plsc.parallel_loop(0, w, step=_LANES, unroll=8)
                def _(i):
                    o = pl.multiple_of(i, _LANES)
                    buf[pl.ds(o, _LANES)] = jnp.exp(buf[pl.ds(o, _LANES)] * s)

            in_copy(0).start()
            if nk > 1:
                in_copy(1).start()
            for k in range(nk):
                in_copy(k).wait()
                compute(k)
                out_copy(k).start()
                if k + 2 < nk:
                    if k >= 1:
                        out_copy(k - 1).wait()
                    in_copy(k + 2).start()
            for k in range(max(0, nk - _NBUF), nk):
                out_copy(k).wait()

        std_lens = (cw,) * _NCHUNK
        last_lens = (cw,) * full_last + (cw_tail,)

        @pl.when(wid < _NUM_WORKERS - 1)
        def _():
            pipeline(wid * c_std, std_lens)

        @pl.when(wid == _NUM_WORKERS - 1)
        def _():
            pipeline(last_start, last_lens)

    return run


def kernel(indices, values, rationality):
    del indices
    run = _build_sc_exp_map(values.shape[0])
    scale = jnp.full((_LANES,), -rationality, dtype=jnp.float32)
    return run(values, scale)
